# Initial kernel scaffold; baseline (speedup 1.0000x reference)
import jax
import jax.numpy as jnp
from jax.experimental import pallas as pl


def kernel(user_emb, item_emb, adj_val, adj_row, adj_col, user, pos, neg):
    raise NotImplementedError("write your pallas kernel here")


# speedup vs baseline: 6.5766x; 6.5766x over previous
import functools

import numpy as np
import jax
import jax.numpy as jnp
from jax import lax
from jax.experimental import pallas as pl
from jax.experimental.pallas import tpu as pltpu
from jax.experimental.pallas import tpu_sc as plsc

NUM_USERS = 50000
NUM_ITEMS = 50000
DIM = 32
BATCH = 4096
N_NODES_T = 100000
NNZ = 1600000
EDGES_PER_SC = NNZ // 2
CHUNK = 128
CHUNKS_PER_SC = EDGES_PER_SC // CHUNK
NC, NS = 2, 16
ROWS_PER_SC = 50000
ROWS_PER_TILE = ROWS_PER_SC // NS
ZROWS = 125
WROWS = 1000
SSL_LAMBDA = 0.1
TEMPERATURE = 0.2
REG_WEIGHT = 1e-4


def _consts():
    kk = jax.random.key(42)
    kb1, kb2, kp, kd1, kd2 = jax.random.split(kk, 5)
    beta_u = jax.random.beta(kb1, 0.2, 0.2, (BATCH, 1)).astype(jnp.float32)
    beta_i = jax.random.beta(kb2, 0.2, 0.2, (BATCH, 1)).astype(jnp.float32)
    perm = jax.random.permutation(kp, BATCH)
    coeff_u = jax.random.dirichlet(kd1, jnp.ones(BATCH)).astype(jnp.float32)
    coeff_p = jax.random.dirichlet(kd2, jnp.ones(BATCH)).astype(jnp.float32)
    return (beta_u.reshape(1, BATCH), beta_i.reshape(1, BATCH),
            perm.astype(jnp.int32), coeff_u.reshape(1, BATCH),
            coeff_p.reshape(1, BATCH))


def _prop_body(table, rows, cols, vals, out,
               colbuf, rowbuf, valbuf, gbuf, zbuf, gsem, ssem, acc):
    c = lax.axis_index("c")
    s = lax.axis_index("s")
    base_edge = c * EDGES_PER_SC
    base_node = c * ROWS_PER_SC

    zero16 = jnp.zeros((16,), jnp.float32)
    for i in range(ZROWS):
        zbuf[i, pl.ds(0, 16)] = zero16
        zbuf[i, pl.ds(16, 16)] = zero16
    for k in range(ROWS_PER_TILE // ZROWS):
        pltpu.sync_copy(zbuf, acc.at[pl.ds(s * ROWS_PER_TILE + k * ZROWS, ZROWS)])
    plsc.subcore_barrier()

    niters = 390 + (s < CHUNKS_PER_SC - 390 * NS).astype(jnp.int32)

    def body(i, _):
        j = s + i * NS
        base = pl.multiple_of(base_edge + j * CHUNK, CHUNK)
        pltpu.sync_copy(cols.at[pl.ds(base, CHUNK)], colbuf)
        pltpu.sync_copy(rows.at[pl.ds(base, CHUNK)], rowbuf.at[0])
        pltpu.sync_copy(vals.at[pl.ds(base, CHUNK)], valbuf)
        pltpu.async_copy(table.at[colbuf], gbuf, gsem).wait()
        for k in range(CHUNK // 16):
            sl = pl.ds(k * 16, 16)
            rowbuf[0, sl] = rowbuf[0, sl] - base_node
        for g in range(CHUNK // 16):
            v16 = valbuf[pl.ds(g * 16, 16)]
            for l in range(16):
                e = g * 16 + l
                v = v16[l]
                gbuf[e, pl.ds(0, 16)] = gbuf[e, pl.ds(0, 16)] * v
                gbuf[e, pl.ds(16, 16)] = gbuf[e, pl.ds(16, 16)] * v
        pltpu.async_copy(gbuf, acc.at[rowbuf.at[0]], ssem, add=True).wait()
        return ()

    lax.fori_loop(0, niters, body, ())
    plsc.subcore_barrier()

    nw = ROWS_PER_SC // WROWS
    for k in range(pl.cdiv(nw, NS)):
        m = s + k * NS

        @pl.when(m < nw)
        def _():
            r0 = pl.multiple_of(m * WROWS, 8)
            pltpu.sync_copy(acc.at[pl.ds(r0, WROWS)],
                            out.at[pl.ds(base_node + r0, WROWS)])


@functools.lru_cache(maxsize=None)
def _make_prop(v_rows):
    mesh = plsc.VectorSubcoreMesh(core_axis_name="c", subcore_axis_name="s", num_cores=NC, num_subcores=NS)
    return pl.kernel(
        _prop_body,
        out_type=jax.ShapeDtypeStruct((N_NODES_T, DIM), jnp.float32),
        mesh=mesh,
        compiler_params=pltpu.CompilerParams(use_tc_tiling_on_sc=False),
        scratch_types=[
            pltpu.VMEM((CHUNK,), jnp.int32),
            pltpu.VMEM((1, CHUNK), jnp.int32),
            pltpu.VMEM((CHUNK,), jnp.float32),
            pltpu.VMEM((CHUNK, DIM), jnp.float32),
            pltpu.VMEM((ZROWS, DIM), jnp.float32),
            pltpu.SemaphoreType.DMA,
            pltpu.SemaphoreType.DMA,
            pltpu.VMEM_SHARED((ROWS_PER_SC, DIM), jnp.float32),
        ],
        name=f"lgconv_prop_{v_rows}",
    )


GB = 6 * BATCH
GCHUNKS = GB // CHUNK
GPER = GCHUNKS // (NC * NS)


def _gather_body(ego, cur1, cur2, gidx, fsum, g0h, idxb, gb0, gb1, gb2, sem):
    c = lax.axis_index("c")
    s = lax.axis_index("s")
    wid = s * NC + c

    def body(k, _):
        m = k * (NC * NS) + wid
        base = pl.multiple_of(m * CHUNK, CHUNK)
        pltpu.sync_copy(gidx.at[pl.ds(base, CHUNK)], idxb)
        pltpu.async_copy(ego.at[idxb], gb0, sem).wait()

        @pl.when(m < 3 * BATCH // CHUNK)
        def _():
            pltpu.sync_copy(gb0, g0h.at[pl.ds(base, CHUNK)])

        pltpu.async_copy(cur1.at[idxb], gb1, sem).wait()
        pltpu.async_copy(cur2.at[idxb], gb2, sem).wait()
        for r in range(CHUNK):
            for h in (0, 16):
                sl = pl.ds(h, 16)
                gb0[r, sl] = gb0[r, sl] + gb1[r, sl] + gb2[r, sl]
        pltpu.sync_copy(gb0, fsum.at[pl.ds(base, CHUNK)])
        return ()

    lax.fori_loop(0, GPER, body, ())


@functools.lru_cache(maxsize=None)
def _make_gather():
    return pl.kernel(
        _gather_body,
        out_type=[jax.ShapeDtypeStruct((GB, DIM), jnp.float32),
                  jax.ShapeDtypeStruct((3 * BATCH, DIM), jnp.float32)],
        mesh=plsc.VectorSubcoreMesh(core_axis_name="c", subcore_axis_name="s", num_cores=NC, num_subcores=NS),
        compiler_params=pltpu.CompilerParams(use_tc_tiling_on_sc=False),
        scratch_types=[
            pltpu.VMEM((CHUNK,), jnp.int32),
            pltpu.VMEM((CHUNK, DIM), jnp.float32),
            pltpu.VMEM((CHUNK, DIM), jnp.float32),
            pltpu.VMEM((CHUNK, DIM), jnp.float32),
            pltpu.SemaphoreType.DMA,
        ],
        name="batch_gather",
    )


def _normalize(x):
    n = jnp.sqrt(jnp.sum(x * x, axis=1, keepdims=True))
    return x / jnp.clip(n, 1e-12, None)


def _softplus(x):
    return jnp.maximum(x, 0.0) + jnp.log1p(jnp.exp(-jnp.abs(x)))


def _dot_nt(a, b):
    return lax.dot_general(a, b, (((1,), (1,)), ((), ())),
                           precision=lax.Precision.HIGHEST,
                           preferred_element_type=jnp.float32)


def _loss_body(fs_ref, g0h_ref, bu_ref, bi_ref, cu_ref, cp_ref, out_ref,
               s_a, s_p, s_sims):
    B = BATCH
    T = TEMPERATURE

    def fs(i):
        return fs_ref[pl.ds(i * B, B), :] * (1.0 / 3.0)

    u_e, pos_e, neg_e = fs(0), fs(1), fs(2)
    u_dis, pos_dis, neg_dis = fs(3), fs(4), fs(5)
    bu = jnp.reshape(bu_ref[...], (B, 1))
    bi = jnp.reshape(bi_ref[...], (B, 1))

    pos_scores = jnp.sum(u_e * pos_e, axis=1)
    neg_scores = jnp.sum(u_e * neg_e, axis=1)
    neg_mix = bi * neg_e + (1.0 - bi) * neg_dis
    neg_mix_scores = jnp.sum(u_e * neg_mix, axis=1)
    bpr_pos = _softplus(-(pos_scores - neg_scores))
    bpr_neg = _softplus(-(pos_scores - neg_mix_scores))
    bim = jnp.mean(bi)
    bum = jnp.mean(bu)
    main = bim * jnp.mean(bpr_pos) + (1.0 - bim) * jnp.mean(bpr_neg)

    u_mix = bu * u_e + (1.0 - bu) * u_dis
    pos_mix = bi * pos_e + (1.0 - bi) * pos_dis
    u_cm = jnp.dot(cu_ref[...], u_e, precision=lax.Precision.HIGHEST,
                   preferred_element_type=jnp.float32)
    pos_cm = jnp.dot(cp_ref[...], pos_e, precision=lax.Precision.HIGHEST,
                     preferred_element_type=jnp.float32)

    u_en, u_mixn, u_disn, u_cmn = map(_normalize, (u_e, u_mix, u_dis, u_cm))
    p_en, p_mixn, p_disn, p_cmn = map(_normalize, (pos_e, pos_mix, pos_dis, pos_cm))

    quads = ((u_en, u_mixn, u_disn, u_cmn), (u_disn, u_mixn, u_en, u_cmn),
             (p_en, p_mixn, p_disn, p_cmn), (p_disn, p_mixn, p_en, p_cmn))
    for n, (a, p, d, cn) in enumerate(quads):
        s_a[n] = a
        s_sims[3 * n + 0, :] = jnp.sum(a * p, axis=1) / T
        s_sims[3 * n + 1, :] = jnp.sum(a * d, axis=1) / T
        s_sims[3 * n + 2, :] = jnp.sum(a * cn, axis=1) / T
    s_p[0] = u_mixn
    s_p[1] = p_mixn

    blk = 256
    nblk = B // blk

    def body(b, tot):
        r0 = b * blk
        out = []
        for n in range(4):
            a_blk = s_a[n, pl.ds(r0, blk), :]
            mat = _dot_nt(a_blk, s_p[n // 2]) / T
            ps = s_sims[3 * n + 0, pl.ds(r0, blk)]
            dsim = s_sims[3 * n + 1, pl.ds(r0, blk)]
            csim = s_sims[3 * n + 2, pl.ds(r0, blk)]
            m = jnp.maximum(jnp.max(mat, axis=1), jnp.maximum(dsim, csim))
            se = jnp.sum(jnp.exp(mat - m[:, None]), axis=1)
            se = se + jnp.exp(dsim - m) + jnp.exp(csim - m)
            out.append(tot[n] + jnp.sum(jnp.log(se) + m - ps))
        return tuple(out)

    z = jnp.float32(0.0)
    t0, t1, t2, t3 = lax.fori_loop(0, nblk, body, (z, z, z, z))

    cl_user = (bum * t0 + (1.0 - bum) * t1) / B
    cl_item = (bim * t2 + (1.0 - bim) * t3) / B

    reg = REG_WEIGHT * jnp.sum(g0h_ref[...] ** 2) / B

    loss = main + SSL_LAMBDA * (cl_user + cl_item) + reg
    out_ref[...] = jnp.full((1, 1), loss, jnp.float32)


_loss_tc = pl.pallas_call(
    _loss_body,
    out_shape=jax.ShapeDtypeStruct((1, 1), jnp.float32),
    scratch_shapes=[
        pltpu.VMEM((4, BATCH, DIM), jnp.float32),
        pltpu.VMEM((2, BATCH, DIM), jnp.float32),
        pltpu.VMEM((12, BATCH), jnp.float32),
    ],
    compiler_params=pltpu.CompilerParams(vmem_limit_bytes=128 * 1024 * 1024),
)


def kernel(user_emb, item_emb, adj_val, adj_row, adj_col, user, pos, neg):
    ego = jnp.concatenate([user_emb, item_emb], axis=0)
    cur1 = _make_prop(NUM_USERS + NUM_ITEMS + 1)(ego, adj_row, adj_col, adj_val)
    cur2 = _make_prop(N_NODES_T)(cur1, adj_row, adj_col, adj_val)

    beta_u, beta_i, perm, coeff_u, coeff_p = _consts()
    user = user.astype(jnp.int32)
    posn = pos.astype(jnp.int32) + NUM_USERS
    negn = neg.astype(jnp.int32) + NUM_USERS
    gidx = jnp.concatenate([user, posn, negn, jnp.take(user, perm),
                            jnp.take(posn, perm), jnp.take(negn, perm)])
    fsum, g0h = _make_gather()(ego, cur1, cur2, gidx)

    loss = _loss_tc(fsum, g0h, beta_u, beta_i, coeff_u, coeff_p)
    return jnp.reshape(loss, ())
